# baseline (speedup 1.0000x reference)
import functools

import jax
import jax.numpy as jnp
from jax.experimental import pallas as pl
from jax.experimental.pallas import tpu as pltpu

N = 4096
IN_FEAT = 128
BLK = 256
GRID = N // BLK


def _fused_kernel(adj_ref, b_ref, w_enc_ref, x_latent_ref, x_recon_ref, ab_acc):
    i = pl.program_id(0)
    ab = jax.lax.dot_general(
        adj_ref[...], b_ref[...], (((1,), (0,)), ((), ())),
        preferred_element_type=jnp.float32,
        precision=jax.lax.Precision.DEFAULT,
    )
    ab_acc[pl.ds(i * BLK, BLK), :] = ab
    x_latent_ref[...] = jax.lax.dot_general(
        ab[:, :IN_FEAT], w_enc_ref[...], (((1,), (0,)), ((), ())),
        preferred_element_type=jnp.float32,
        precision=jax.lax.Precision.DEFAULT,
    )

    @pl.when(i == GRID - 1)
    def _():
        a = ab_acc[:, :IN_FEAT]
        y = ab_acc[:, IN_FEAT:]
        m = jax.lax.dot_general(
            w_enc_ref[...], y, (((1,), (0,)), ((), ())),
            preferred_element_type=jnp.float32,
            precision=jax.lax.Precision.DEFAULT,
        )
        x_recon_ref[...] = jax.lax.dot_general(
            a, m, (((1,), (0,)), ((), ())),
            preferred_element_type=jnp.float32,
            precision=jax.lax.Precision.DEFAULT,
        )


@jax.jit
def _run(feat, adj, W_enc, W_dec):
    b = jnp.concatenate([feat, W_dec], axis=1)
    x_latent, x_recon = pl.pallas_call(
        _fused_kernel,
        grid=(GRID,),
        in_specs=[
            pl.BlockSpec((BLK, N), lambda i: (i, 0)),
            pl.BlockSpec((N, 2 * IN_FEAT), lambda i: (0, 0)),
            pl.BlockSpec((IN_FEAT, N), lambda i: (0, 0)),
        ],
        out_specs=[
            pl.BlockSpec((BLK, N), lambda i: (i, 0)),
            pl.BlockSpec((N, IN_FEAT), lambda i: (0, 0)),
        ],
        out_shape=[
            jax.ShapeDtypeStruct((N, N), jnp.float32),
            jax.ShapeDtypeStruct((N, IN_FEAT), jnp.float32),
        ],
        scratch_shapes=[pltpu.VMEM((N, 2 * IN_FEAT), jnp.float32)],
    )(adj, b, W_enc)
    return x_latent, x_recon


def kernel(feat, adj, W_enc, W_dec):
    return _run(feat, adj, W_enc, W_dec)


# speedup vs baseline: 1.0575x; 1.0575x over previous
import functools

import jax
import jax.numpy as jnp
from jax.experimental import pallas as pl
from jax.experimental.pallas import tpu as pltpu

N = 4096
IN_FEAT = 128
BLK = 512
GRID = N // BLK


def _fused_kernel(adj_ref, b_ref, w_enc_ref, x_latent_ref, x_recon_ref, ab_acc):
    i = pl.program_id(0)
    ab = jax.lax.dot_general(
        adj_ref[...], b_ref[...], (((1,), (0,)), ((), ())),
        preferred_element_type=jnp.float32,
        precision=jax.lax.Precision.DEFAULT,
    )
    ab_acc[pl.ds(i * BLK, BLK), :] = ab
    x_latent_ref[...] = jax.lax.dot_general(
        ab[:, :IN_FEAT], w_enc_ref[...], (((1,), (0,)), ((), ())),
        preferred_element_type=jnp.float32,
        precision=jax.lax.Precision.DEFAULT,
    )

    @pl.when(i == GRID - 1)
    def _():
        a = ab_acc[:, :IN_FEAT]
        y = ab_acc[:, IN_FEAT:]
        m = jax.lax.dot_general(
            w_enc_ref[...], y, (((1,), (0,)), ((), ())),
            preferred_element_type=jnp.float32,
            precision=jax.lax.Precision.DEFAULT,
        )
        x_recon_ref[...] = jax.lax.dot_general(
            a, m, (((1,), (0,)), ((), ())),
            preferred_element_type=jnp.float32,
            precision=jax.lax.Precision.DEFAULT,
        )


@jax.jit
def _run(feat, adj, W_enc, W_dec):
    b = jnp.concatenate([feat, W_dec], axis=1)
    x_latent, x_recon = pl.pallas_call(
        _fused_kernel,
        grid=(GRID,),
        in_specs=[
            pl.BlockSpec((BLK, N), lambda i: (i, 0)),
            pl.BlockSpec((N, 2 * IN_FEAT), lambda i: (0, 0)),
            pl.BlockSpec((IN_FEAT, N), lambda i: (0, 0)),
        ],
        out_specs=[
            pl.BlockSpec((BLK, N), lambda i: (i, 0)),
            pl.BlockSpec((N, IN_FEAT), lambda i: (0, 0)),
        ],
        out_shape=[
            jax.ShapeDtypeStruct((N, N), jnp.float32),
            jax.ShapeDtypeStruct((N, IN_FEAT), jnp.float32),
        ],
        scratch_shapes=[pltpu.VMEM((N, 2 * IN_FEAT), jnp.float32)],
    )(adj, b, W_enc)
    return x_latent, x_recon


def kernel(feat, adj, W_enc, W_dec):
    return _run(feat, adj, W_enc, W_dec)
